# baseline (speedup 1.0000x reference)
import jax
import jax.numpy as jnp
from jax import lax
from jax.experimental import pallas as pl
from jax.experimental.pallas import tpu as pltpu
from jax.experimental.pallas import tpu_sc as plsc

_B = 8
_C_AB = 6
_C_B = 3
_H = 512
_W = 512
_S = 64
_PRO = 62
_R = 64
_NEG = -3.4e38


def _sc_body(real_AB, fake_AB, score_map, real_B, fake_B,
             real_Br, fake_Br, patch, rbl,
             score_v, row_v, crop_v, prow_v, frow_v, out16_v):
    wid = lax.axis_index("s") * 2 + lax.axis_index("c")
    b = wid // 4
    role = wid % 4
    iota = lax.iota(jnp.int32, 16)

    pltpu.sync_copy(score_map.at[b, 0], score_v)

    def row_step(r, carry):
        rmax, ridx = carry
        for k in range(4):
            v = score_v[r, pl.ds(16 * k, 16)]
            if k == 3:
                v = jnp.where(iota < _PRO - 48, v, jnp.float32(_NEG))
            cm = jnp.max(v)
            pos = jnp.min(jnp.where(v == cm, iota, 16))
            flat = r * _PRO + 16 * k + pos
            upd = cm > rmax
            rmax = jnp.where(upd, cm, rmax)
            ridx = jnp.where(upd, flat, ridx)
        return rmax, ridx

    rmax, ridx = lax.fori_loop(0, _PRO, row_step,
                               (jnp.float32(_NEG), jnp.int32(0)))
    valid = rmax > 0.0
    rb = jnp.where(valid, ridx // _PRO, 0) * 6 + 70
    cb = jnp.where(valid, ridx % _PRO, 0) * 6 + 70

    def do_crops(src, dst):
        for c in range(_C_B):
            pltpu.sync_copy(src.at[b, c, pl.ds(rb, _R), :], row_v)

            def per_row(r, _):
                rvec = jnp.full((16,), r, jnp.int32)
                for k in range(4):
                    cidx = cb + 16 * k + iota
                    v = plsc.load_gather(row_v, [rvec, cidx])
                    crop_v[r, pl.ds(16 * k, 16)] = v
                return 0

            lax.fori_loop(0, _R, per_row, 0)
            pltpu.sync_copy(crop_v, dst.at[b, c])

    @pl.when(role == 0)
    def _():
        do_crops(real_B, real_Br)

    @pl.when(role == 1)
    def _():
        do_crops(fake_B, fake_Br)

    def do_patch(c_lo):
        for c in range(c_lo, c_lo + 3):
            pltpu.sync_copy(real_AB.at[b, c, rb, :], prow_v)
            pltpu.sync_copy(fake_AB.at[b, c, rb, :], frow_v)
            lane = (cb // 16) * 16 + iota
            pv = plsc.load_gather(prow_v, [lane])
            fv = plsc.load_gather(frow_v, [lane])
            plsc.store_scatter(prow_v, [lane], jnp.where(lane == cb, fv, pv))
            pltpu.sync_copy(prow_v, patch.at[b * _C_AB + c])

    @pl.when(role == 2)
    def _():
        do_patch(0)

    @pl.when(role == 3)
    def _():
        do_patch(3)
        out16_v[...] = jnp.full((16,), rb, jnp.int32)
        pltpu.sync_copy(out16_v, rbl.at[b])


_sc_call = pl.kernel(
    _sc_body,
    out_type=(
        jax.ShapeDtypeStruct((_B, _C_B, _R, _R), jnp.float32),
        jax.ShapeDtypeStruct((_B, _C_B, _R, _R), jnp.float32),
        jax.ShapeDtypeStruct((_B * _C_AB, _W), jnp.float32),
        jax.ShapeDtypeStruct((_B, 16), jnp.int32),
    ),
    mesh=plsc.VectorSubcoreMesh(core_axis_name="c", subcore_axis_name="s"),
    compiler_params=pltpu.CompilerParams(use_tc_tiling_on_sc=False,
                                         needs_layout_passes=False),
    scratch_types=[
        pltpu.VMEM((_S, _S), jnp.float32),
        pltpu.VMEM((_R, _W), jnp.float32),
        pltpu.VMEM((_R, _R), jnp.float32),
        pltpu.VMEM((_W,), jnp.float32),
        pltpu.VMEM((_W,), jnp.float32),
        pltpu.VMEM((16,), jnp.int32),
    ],
)


def _tc_body(rbl_ref, in_ref, patch_ref, out_ref):
    out_ref[...] = in_ref[...]
    i = pl.program_id(0)
    rb = rbl_ref[i // _C_AB, 0]
    out_ref[pl.ds(rb, 1), :] = patch_ref[0]


_tc_call = pl.pallas_call(
    _tc_body,
    grid=(_B * _C_AB,),
    in_specs=[
        pl.BlockSpec(memory_space=pltpu.SMEM),
        pl.BlockSpec((_H, _W), lambda i: (i, 0)),
        pl.BlockSpec((1, 1, _W), lambda i: (i, 0, 0)),
    ],
    out_specs=pl.BlockSpec((_H, _W), lambda i: (i, 0)),
    out_shape=jax.ShapeDtypeStruct((_B * _C_AB * _H, _W), jnp.float32),
    compiler_params=pltpu.CompilerParams(
        dimension_semantics=("arbitrary",),
    ),
)


def kernel(real_AB, fake_AB, score_map, real_B, fake_B):
    real_Br, fake_Br, patch, rbl = _sc_call(
        real_AB, fake_AB, score_map, real_B, fake_B)
    flat_in = real_AB.reshape(_B * _C_AB * _H, _W)
    patch3 = patch.reshape(_B * _C_AB, 1, _W)
    fake_ABm = _tc_call(rbl, flat_in, patch3).reshape(_B, _C_AB, _H, _W)
    return fake_ABm, real_Br, fake_Br


# speedup vs baseline: 1.2158x; 1.2158x over previous
import jax
import jax.numpy as jnp
from jax import lax
from jax.experimental import pallas as pl
from jax.experimental.pallas import tpu as pltpu
from jax.experimental.pallas import tpu_sc as plsc

_B = 8
_C_AB = 6
_C_B = 3
_H = 512
_W = 512
_S = 64
_PRO = 62
_R = 64
_NEG = -3.4e38


def _sc_loc_body(score_map, rcl, score_v, out_v):
    wid = lax.axis_index("s") * 2 + lax.axis_index("c")
    iota = lax.iota(jnp.int32, 16)

    @pl.when(wid < _B)
    def _():
        b = wid
        pltpu.sync_copy(score_map.at[b, 0], score_v)

        def row_step(r, carry):
            rmax, ridx = carry
            for k in range(4):
                v = score_v[r, pl.ds(16 * k, 16)]
                if k == 3:
                    v = jnp.where(iota < _PRO - 48, v, jnp.float32(_NEG))
                cm = jnp.max(v)
                pos = jnp.min(jnp.where(v == cm, iota, 16))
                flat = r * _PRO + 16 * k + pos
                upd = cm > rmax
                rmax = jnp.where(upd, cm, rmax)
                ridx = jnp.where(upd, flat, ridx)
            return rmax, ridx

        rmax, ridx = lax.fori_loop(0, _PRO, row_step,
                                   (jnp.float32(_NEG), jnp.int32(0)))
        valid = rmax > 0.0
        rb = jnp.where(valid, ridx // _PRO, 0) * 6 + 70
        cb = jnp.where(valid, ridx % _PRO, 0) * 6 + 70
        out_v[...] = jnp.where(iota < 8, rb, cb)
        pltpu.sync_copy(out_v, rcl.at[b])


_sc_loc = pl.kernel(
    _sc_loc_body,
    out_type=jax.ShapeDtypeStruct((_B, 16), jnp.int32),
    mesh=plsc.VectorSubcoreMesh(core_axis_name="c", subcore_axis_name="s"),
    compiler_params=pltpu.CompilerParams(use_tc_tiling_on_sc=False,
                                         needs_layout_passes=False),
    scratch_types=[
        pltpu.VMEM((_S, _S), jnp.float32),
        pltpu.VMEM((16,), jnp.int32),
    ],
)


def _tc_crop_body(rcl_ref, rB_ref, fB_ref, rBr_ref, fBr_ref, sr, sf):
    t = pl.program_id(2)
    sr[pl.ds(t * 8, 8), :] = rB_ref[0, 0]
    sf[pl.ds(t * 8, 8), :] = fB_ref[0, 0]

    @pl.when(t == 8)
    def _():
        b = pl.program_id(0)
        rb = rcl_ref[b, 0]
        cb = rcl_ref[b, 8]
        roff = rb - (rb // 8) * 8

        def win(s):
            v = pltpu.roll(s[...], lax.rem(72 - roff, 72), axis=0)
            v = pltpu.roll(v, _W - cb, axis=1)
            return v[:_R, :_R]

        rBr_ref[0, 0] = win(sr)
        fBr_ref[0, 0] = win(sf)


_tc_crop = pl.pallas_call(
    _tc_crop_body,
    grid_spec=pltpu.PrefetchScalarGridSpec(
        num_scalar_prefetch=1,
        grid=(_B, _C_B, 9),
        in_specs=[
            pl.BlockSpec((1, 1, 8, _W),
                         lambda b, c, t, rcl: (b, c, rcl[b, 0] // 8 + t, 0)),
            pl.BlockSpec((1, 1, 8, _W),
                         lambda b, c, t, rcl: (b, c, rcl[b, 0] // 8 + t, 0)),
        ],
        out_specs=[
            pl.BlockSpec((1, 1, _R, _R), lambda b, c, t, rcl: (b, c, 0, 0)),
            pl.BlockSpec((1, 1, _R, _R), lambda b, c, t, rcl: (b, c, 0, 0)),
        ],
        scratch_shapes=[
            pltpu.VMEM((72, _W), jnp.float32),
            pltpu.VMEM((72, _W), jnp.float32),
        ],
    ),
    out_shape=(
        jax.ShapeDtypeStruct((_B, _C_B, _R, _R), jnp.float32),
        jax.ShapeDtypeStruct((_B, _C_B, _R, _R), jnp.float32),
    ),
    compiler_params=pltpu.CompilerParams(
        dimension_semantics=("arbitrary", "arbitrary", "arbitrary"),
    ),
)


def _tc_copy_body(rcl_ref, rAB_ref, fABt_ref, out_ref):
    out_ref[...] = rAB_ref[...]
    b = pl.program_id(0)
    rb = rcl_ref[b, 0]
    cb = rcl_ref[b, 8]
    rb8 = (rb // 8) * 8
    r8 = lax.broadcasted_iota(jnp.int32, (8, _W), 0)
    c8 = lax.broadcasted_iota(jnp.int32, (8, _W), 1)
    m8 = (r8 == rb - rb8) & (c8 == cb)
    sx = rAB_ref[0, 0, pl.ds(rb8, 8), :]
    out_ref[0, 0, pl.ds(rb8, 8), :] = jnp.where(m8, fABt_ref[0, 0], sx)


_tc_copy = pl.pallas_call(
    _tc_copy_body,
    grid_spec=pltpu.PrefetchScalarGridSpec(
        num_scalar_prefetch=1,
        grid=(_B, _C_AB),
        in_specs=[
            pl.BlockSpec((1, 1, _H, _W), lambda b, c, rcl: (b, c, 0, 0)),
            pl.BlockSpec((1, 1, 8, _W),
                         lambda b, c, rcl: (b, c, rcl[b, 0] // 8, 0)),
        ],
        out_specs=pl.BlockSpec((1, 1, _H, _W), lambda b, c, rcl: (b, c, 0, 0)),
    ),
    out_shape=jax.ShapeDtypeStruct((_B, _C_AB, _H, _W), jnp.float32),
    compiler_params=pltpu.CompilerParams(
        dimension_semantics=("arbitrary", "arbitrary"),
    ),
)


def kernel(real_AB, fake_AB, score_map, real_B, fake_B):
    rcl = _sc_loc(score_map)
    real_Br, fake_Br = _tc_crop(rcl, real_B, fake_B)
    fake_ABm = _tc_copy(rcl, real_AB, fake_AB)
    return fake_ABm, real_Br, fake_Br
